# baseline (speedup 1.0000x reference)
import functools

import jax
import jax.numpy as jnp
from jax import lax
from jax.experimental import pallas as pl
from jax.experimental.pallas import tpu as pltpu
from jax.experimental.pallas import tpu_sc as plsc

N = 100000
E = 1600000
IN = 32
H = 64
G = 64

L = 16
F = 32
NTILES = 16
BN = 2048
NP = 49 * BN
NP4 = NP // 4
BP4 = BN // 4
EP = 1638400
CH = 1280
KSUB = CH // 128
RPT = NP // NTILES
STG = RPT // 16
NSEG = G + 1
PNT = NP // 32
PNR4 = PNT // 4

_f32 = jnp.float32
_bf16 = jnp.bfloat16
_i32 = jnp.int32


def _fill(ref, rows, vec):
    def body(i, _):
        ref[i] = vec
        return 0
    lax.fori_loop(0, rows, body, 0)


def _zero_accum(stage, accum, s):
    _fill(stage, STG, jnp.zeros((F,), _bf16))
    rz = s * RPT
    for r in range(16):
        pltpu.sync_copy(stage, accum.at[pl.ds(rz + r * STG, STG)])


def _write_out(stage, accum, out, s):
    rz = s * RPT
    for r in range(16):
        pltpu.sync_copy(accum.at[pl.ds(rz + r * STG, STG)], stage)
        pltpu.sync_copy(stage, out.at[pl.ds(rz + r * STG, STG)])


def _scan_edges(src2, dst2, table, sidx, didx, rows, accum,
                semg, sems, wid, nch):
    def chunk(j, _):
        rb = wid * (nch * KSUB) + j * KSUB
        pltpu.sync_copy(src2.at[pl.ds(rb, KSUB)], sidx)
        pltpu.sync_copy(dst2.at[pl.ds(rb, KSUB)], didx)
        gd = [pltpu.async_copy(table.at[sidx.at[k]],
                               rows.at[pl.ds(k * 128, 128)], semg)
              for k in range(KSUB)]
        sd = []
        for k in range(KSUB):
            gd[k].wait()
            sd.append(pltpu.async_copy(rows.at[pl.ds(k * 128, 128)],
                                       accum.at[didx.at[k]], sems, add=True))
        for d in sd:
            d.wait()
        return 0
    lax.fori_loop(0, nch, chunk, 0)


def _agg_pass(src2, dst2, table, out, sidx, didx, rows, stage, accum,
              semg, sems, s, wid, nch):
    _zero_accum(stage, accum, s)
    plsc.subcore_barrier()
    _scan_edges(src2, dst2, table, sidx, didx, rows, accum,
                semg, sems, wid, nch)
    plsc.subcore_barrier()
    _write_out(stage, accum, out, s)
    plsc.subcore_barrier()


_AGG_SCRATCH = [
    pltpu.VMEM((KSUB, 128), _i32),
    pltpu.VMEM((KSUB, 128), _i32),
    pltpu.VMEM((CH, F), _bf16),
    pltpu.VMEM((STG, F), _bf16),
    pltpu.VMEM_SHARED((NP, F), _bf16),
    pltpu.SemaphoreType.DMA,
    pltpu.SemaphoreType.DMA,
]


@functools.partial(
    pl.kernel,
    out_type=(jax.ShapeDtypeStruct((NP, F), _bf16),
              jax.ShapeDtypeStruct((NP, F), _bf16)),
    mesh=plsc.VectorSubcoreMesh(core_axis_name="c", subcore_axis_name="s"),
    scratch_types=[
        pltpu.VMEM((KSUB, 128), _i32),
        pltpu.VMEM((128, F), _bf16),
        pltpu.VMEM((STG, F), _bf16),
        pltpu.VMEM_SHARED((NP, F), _bf16),
        pltpu.SemaphoreType.DMA,
    ],
    compiler_params=pltpu.CompilerParams(use_tc_tiling_on_sc=False),
)
def _deg_kernel(dst2, o0, o1, didx, ones_v, stage, accum, sem):
    c = lax.axis_index("c")
    s = lax.axis_index("s")
    w = c * NTILES + s
    _fill(ones_v, 128, jnp.ones((F,), _bf16))
    _zero_accum(stage, accum, s)
    plsc.subcore_barrier()

    def chunk(j, _):
        rb = w * (EP // 32 // 128) + j * KSUB
        pltpu.sync_copy(dst2.at[pl.ds(rb, KSUB)], didx)
        sd = [pltpu.async_copy(ones_v, accum.at[didx.at[k]], sem, add=True)
              for k in range(KSUB)]
        for d in sd:
            d.wait()
        return 0
    lax.fori_loop(0, EP // 32 // CH, chunk, 0)
    plsc.subcore_barrier()

    @pl.when(c == 0)
    def _():
        _write_out(stage, accum, o0, s)

    @pl.when(c == 1)
    def _():
        _write_out(stage, accum, o1, s)


@functools.partial(
    pl.kernel,
    out_type=(jax.ShapeDtypeStruct((NP, F), _bf16),
              jax.ShapeDtypeStruct((NP, F), _bf16)),
    mesh=plsc.VectorSubcoreMesh(core_axis_name="c", subcore_axis_name="s"),
    scratch_types=list(_AGG_SCRATCH),
    compiler_params=pltpu.CompilerParams(use_tc_tiling_on_sc=False),
)
def _agg1_kernel(src2, dst2, xsb, t0, t1,
                 sidx, didx, rows, stage, accum, semg, sems):
    c = lax.axis_index("c")
    s = lax.axis_index("s")
    w = c * NTILES + s
    _zero_accum(stage, accum, s)
    plsc.subcore_barrier()
    _scan_edges(src2, dst2, xsb, sidx, didx, rows, accum,
                semg, sems, w, EP // 32 // CH)
    plsc.subcore_barrier()

    @pl.when(c == 0)
    def _():
        _write_out(stage, accum, t0, s)

    @pl.when(c == 1)
    def _():
        _write_out(stage, accum, t1, s)


@functools.partial(
    pl.kernel,
    out_type=(jax.ShapeDtypeStruct((NP, F), _bf16),
              jax.ShapeDtypeStruct((NP, F), _bf16)),
    mesh=plsc.VectorSubcoreMesh(core_axis_name="c", subcore_axis_name="s"),
    scratch_types=list(_AGG_SCRATCH),
    compiler_params=pltpu.CompilerParams(use_tc_tiling_on_sc=False),
)
def _agg2_kernel(src2, dst2, g0, g1, u0, u1,
                 sidx, didx, rows, stage, accum, semg, sems):
    c = lax.axis_index("c")
    s = lax.axis_index("s")

    @pl.when(c == 0)
    def _():
        _agg_pass(src2, dst2, g0, u0, sidx, didx, rows, stage, accum,
                  semg, sems, s, s, EP // NTILES // CH)

    @pl.when(c == 1)
    def _():
        _agg_pass(src2, dst2, g1, u1, sidx, didx, rows, stage, accum,
                  semg, sems, s, s, EP // NTILES // CH)


@functools.partial(
    pl.kernel,
    out_type=(jax.ShapeDtypeStruct((32, NSEG, L), _f32),
              jax.ShapeDtypeStruct((32, NSEG, L), _f32)),
    mesh=plsc.VectorSubcoreMesh(core_axis_name="c", subcore_axis_name="s"),
    scratch_types=[
        pltpu.VMEM((PNT // 8, 128), _f32),
        pltpu.VMEM((PNT,), _i32),
        pltpu.VMEM((NSEG, L), _f32),
        pltpu.VMEM((NSEG, L), _f32),
    ],
    compiler_params=pltpu.CompilerParams(use_tc_tiling_on_sc=False,
                                         needs_layout_passes=False),
)
def _pool_kernel(zp, bpad, sums_o, cnts_o, zv, bv, sacc, cacc):
    c = lax.axis_index("c")
    s = lax.axis_index("s")
    w = c * NTILES + s
    _fill(sacc, NSEG, jnp.zeros((L,), _f32))
    _fill(cacc, NSEG, jnp.zeros((L,), _f32))
    pltpu.sync_copy(zp.at[pl.ds(w * (PNT // 8), PNT // 8)], zv)
    pltpu.sync_copy(bpad.at[pl.ds(w * PNT, PNT)], bv)
    lanes = lax.iota(_i32, L)
    ones16 = jnp.ones((L,), _f32)

    def grp(g, _):
        nloc = g * L + lanes
        ridx = nloc >> 3
        lidx = (nloc & 7) * L
        z16 = plsc.load_gather(zv, [ridx, lidx])
        b16 = bv[pl.ds(g * L, L)]
        plsc.addupdate_scatter(sacc, [b16, lanes], z16)
        plsc.addupdate_scatter(cacc, [b16, lanes], ones16)
        return 0
    lax.fori_loop(0, PNT // L, grp, 0)
    pltpu.sync_copy(sacc, sums_o.at[w])
    pltpu.sync_copy(cacc, cnts_o.at[w])


def _prep_body(p0, p1, x4, dinv_o, xsb_o):
    deg = 1.0 + p0[...].astype(_f32) + p1[...].astype(_f32)
    dinv4 = lax.rsqrt(deg)
    dinv_o[...] = dinv4
    xsb_o[...] = (x4[...] * dinv4).astype(_bf16)


def _layer1_body(tp0, tp1, xsb, dinv, m1b, b1p, g0_o, g1_o):
    dinv4 = dinv[...]
    t = tp0[...].astype(_f32) + tp1[...].astype(_f32)
    agg = (t + xsb[...].astype(_f32)) * dinv4
    m = m1b[...]
    b = b1p[...]
    for k, out in enumerate((g0_o, g1_o)):
        h = jnp.dot(agg, m[k], preferred_element_type=_f32) + b[k:k + 1, :]
        out[...] = (jnp.maximum(h, 0.0) * dinv4).astype(_bf16)


def _layer2_body(u0, u1, g0, g1, dinv, m2b, b2p, mzb, z_o):
    dinv4 = dinv[...]
    aggs = [(u[...].astype(_f32) + g[...].astype(_f32)) * dinv4
            for u, g in ((u0, g0), (u1, g1))]
    m = m2b[...]
    b = b2p[...]
    mzv = mzb[...]
    zp = jnp.zeros((BP4, 128), _f32)
    for k in range(2):
        h = b[k:k + 1, :]
        for s in range(2):
            h = h + jnp.dot(aggs[s], m[s, k], preferred_element_type=_f32)
        h = jnp.maximum(h, 0.0)
        zp = zp + jnp.dot(h, mzv[k], preferred_element_type=_f32)
    z_o[...] = zp


def _combine_body(sums, cnts, lb, out_o):
    ssum = jnp.sum(sums[...][:G, :], axis=1, keepdims=True)
    csum = jnp.sum(cnts[...][:G, :], axis=1, keepdims=True)
    out_o[...] = ssum / jnp.maximum(csum, 1.0) + lb[...]


def _pk_spec():
    return pl.BlockSpec((BP4, 128), lambda i: (i, 0))


def kernel(x, edge_index, batch, W1, b1, W2, b2, lin_W, lin_b):
    pad_row = jnp.arange(128, dtype=_i32)
    npad = (EP - E) // 128
    src2 = jnp.concatenate(
        [edge_index[0].reshape(E // 128, 128),
         jnp.broadcast_to(pad_row, (npad, 128))])
    dst2 = jnp.concatenate(
        [edge_index[1].reshape(E // 128, 128),
         jnp.broadcast_to(N + pad_row, (npad, 128))])
    bpad = jnp.concatenate([batch, jnp.full((NP - N,), G, _i32)])
    x4 = jnp.concatenate([x, jnp.zeros((NP - N, IN), _f32)]).reshape(NP4, 128)

    eye4 = jnp.eye(4, dtype=_f32)

    def bd4(w32):
        return jnp.einsum("pq,ij->piqj", eye4, w32).reshape(128, 128)

    m1b = jnp.stack([bd4(W1[:, 32 * k:32 * k + 32]) for k in range(2)])
    m2b = jnp.stack([jnp.stack([bd4(W2[32 * s:32 * s + 32, 32 * k:32 * k + 32])
                                for k in range(2)]) for s in range(2)])
    mzb = jnp.stack([jnp.einsum("pq,i,j->piqj", eye4,
                                lin_W[32 * k:32 * k + 32, 0],
                                jnp.ones((F,), _f32)).reshape(128, 128)
                     for k in range(2)])
    b1p = jnp.broadcast_to(b1.reshape(2, 1, F), (2, 4, F)).reshape(2, 128)
    b2p = jnp.broadcast_to(b2.reshape(2, 1, F), (2, 4, F)).reshape(2, 128)

    p0, p1 = _deg_kernel(dst2)

    grid = (NP4 // BP4,)
    dinv4, xsb = pl.pallas_call(
        _prep_body, grid=grid,
        in_specs=[_pk_spec()] * 3,
        out_specs=[_pk_spec()] * 2,
        out_shape=[jax.ShapeDtypeStruct((NP4, 128), _f32),
                   jax.ShapeDtypeStruct((NP4, 128), _bf16)],
    )(p0.reshape(NP4, 128), p1.reshape(NP4, 128), x4)

    tp0, tp1 = _agg1_kernel(src2, dst2, xsb.reshape(NP, F))

    g0b, g1b = pl.pallas_call(
        _layer1_body, grid=grid,
        in_specs=[_pk_spec()] * 4 + [
            pl.BlockSpec((2, 128, 128), lambda i: (0, 0, 0)),
            pl.BlockSpec((2, 128), lambda i: (0, 0))],
        out_specs=[_pk_spec()] * 2,
        out_shape=[jax.ShapeDtypeStruct((NP4, 128), _bf16)] * 2,
    )(tp0.reshape(NP4, 128), tp1.reshape(NP4, 128), xsb, dinv4, m1b, b1p)

    u0b, u1b = _agg2_kernel(src2, dst2, g0b.reshape(NP, F), g1b.reshape(NP, F))

    zp4 = pl.pallas_call(
        _layer2_body, grid=grid,
        in_specs=[_pk_spec()] * 5 + [
            pl.BlockSpec((2, 2, 128, 128), lambda i: (0, 0, 0, 0)),
            pl.BlockSpec((2, 128), lambda i: (0, 0)),
            pl.BlockSpec((2, 128, 128), lambda i: (0, 0, 0))],
        out_specs=_pk_spec(),
        out_shape=jax.ShapeDtypeStruct((NP4, 128), _f32),
    )(u0b.reshape(NP4, 128), u1b.reshape(NP4, 128), g0b, g1b, dinv4,
      m2b, b2p, mzb)

    sums, cnts = _pool_kernel(zp4, bpad)
    sums = sums.transpose(1, 0, 2).reshape(NSEG, 512)
    cnts = cnts.transpose(1, 0, 2).reshape(NSEG, 512)

    out = pl.pallas_call(
        _combine_body, grid=(1,),
        in_specs=[pl.BlockSpec((NSEG, 512), lambda i: (0, 0)),
                  pl.BlockSpec((NSEG, 512), lambda i: (0, 0)),
                  pl.BlockSpec((1, 1), lambda i: (0, 0))],
        out_specs=pl.BlockSpec((G, 1), lambda i: (0, 0)),
        out_shape=jax.ShapeDtypeStruct((G, 1), _f32),
    )(sums, cnts, lin_b.reshape(1, 1))
    return out

NP8 = NP // 8
BP8 = BN // 8
PNR8 = PNT // 8


def _fe(w):
    return lax.bitcast_convert_type(w << 16, _f32)


def _fo(w):
    return lax.bitcast_convert_type(w & jnp.int32(-65536), _f32)


def _bfbits(x):
    b = lax.bitcast_convert_type(x, _i32)
    return (b + 0x7FFF + ((b >> 16) & 1)) >> 16


def _pack(lo, hi):
    return (_bfbits(lo) & 0xFFFF) | (_bfbits(hi) << 16)


def _prep_body(p0, p1, xe, xo, dinv_o, xs_o):
    deg = 1.0 + _fe(p0[...]) + _fe(p1[...])
    dinv8 = lax.rsqrt(deg)
    dinv_o[...] = dinv8
    xs_o[...] = _pack(xe[...] * dinv8, xo[...] * dinv8)


def _layer1_body(tp0, tp1, xs, dinv, m1w, b1p, g0_o, g1_o):
    dinv8 = dinv[...]
    xsw = xs[...]
    te = _fe(tp0[...]) + _fe(tp1[...])
    to = _fo(tp0[...]) + _fo(tp1[...])
    agg = ((te + _fe(xsw)) * dinv8, (to + _fo(xsw)) * dinv8)
    m = m1w[...]
    b = b1p[...]
    for k, out in enumerate((g0_o, g1_o)):
        h = []
        for r in range(2):
            hr = b[k, r:r + 1, :]
            for sg in range(2):
                hr = hr + jnp.dot(agg[sg], m[k, r, sg],
                                  preferred_element_type=_f32)
            h.append(jnp.maximum(hr, 0.0) * dinv8)
        out[...] = _pack(h[0], h[1])


def _layer2_body(u0, u1, g0, g1, dinv, m2w, b2p, mzw, z_o):
    dinv8 = dinv[...]
    uws = (u0[...], u1[...])
    gws = (g0[...], g1[...])
    aggs = []
    for k in range(2):
        aggs.append((_fe(uws[k]) + _fe(gws[k])) * dinv8)
        aggs.append((_fo(uws[k]) + _fo(gws[k])) * dinv8)
    m = m2w[...]
    b = b2p[...]
    mzv = mzw[...]
    z8 = jnp.zeros((BP8, 128), _f32)
    for k in range(2):
        for r in range(2):
            h = b[k, r:r + 1, :]
            for sg in range(4):
                h = h + jnp.dot(aggs[sg], m[k, r, sg],
                                preferred_element_type=_f32)
            h = jnp.maximum(h, 0.0)
            z8 = z8 + jnp.dot(h, mzv[k, r], preferred_element_type=_f32)
    z_o[...] = z8


def _combine_body(sums, cnts, lb, out_o):
    ssum = jnp.sum(sums[...][:G, :], axis=1, keepdims=True)
    csum = jnp.sum(cnts[...][:G, :], axis=1, keepdims=True)
    out_o[...] = ssum / jnp.maximum(csum, 1.0) + lb[...]


def _pk_spec():
    return pl.BlockSpec((BP8, 128), lambda i: (i, 0))


def _c2p(a):
    return lax.bitcast_convert_type(a.reshape(NP, 16, 2), _i32).reshape(
        NP8, 128)


def _p2c(w):
    return lax.bitcast_convert_type(w.reshape(NP, 16), _bf16).reshape(NP, F)


def _rep8(v16):
    return jnp.broadcast_to(v16.reshape(1, 16), (8, 16)).reshape(128)


def kernel(x, edge_index, batch, W1, b1, W2, b2, lin_W, lin_b):
    pad_row = jnp.arange(128, dtype=_i32)
    npad = (EP - E) // 128
    src2 = jnp.concatenate(
        [edge_index[0].reshape(E // 128, 128),
         jnp.broadcast_to(pad_row, (npad, 128))])
    dst2 = jnp.concatenate(
        [edge_index[1].reshape(E // 128, 128),
         jnp.broadcast_to(N + pad_row, (npad, 128))])
    bpad = jnp.concatenate([batch, jnp.full((NP - N,), G, _i32)])
    xpad = jnp.concatenate([x, jnp.zeros((NP - N, IN), _f32)])
    xe = xpad[:, 0::2].reshape(NP8, 128)
    xo = xpad[:, 1::2].reshape(NP8, 128)

    eye8 = jnp.eye(8, dtype=_f32)

    def bd8(w16):
        return jnp.einsum("pq,ij->piqj", eye8, w16).reshape(128, 128)

    m1w = jnp.stack([
        jnp.stack([
            jnp.stack([bd8(W1[sg::2, 32 * k + r:32 * k + 32:2])
                       for sg in range(2)])
            for r in range(2)])
        for k in range(2)])
    m2w = jnp.stack([
        jnp.stack([
            jnp.stack([bd8(W2[32 * (sg // 2) + (sg % 2):32 * (sg // 2) + 32:2,
                              32 * k + r:32 * k + 32:2])
                       for sg in range(4)])
            for r in range(2)])
        for k in range(2)])
    mzw = jnp.stack([
        jnp.stack([
            jnp.einsum("pq,i,j->piqj", eye8,
                       lin_W[32 * k + r:32 * k + 32:2, 0],
                       jnp.ones((16,), _f32)).reshape(128, 128)
            for r in range(2)])
        for k in range(2)])
    b1p = jnp.stack([jnp.stack([_rep8(b1[32 * k + r:32 * k + 32:2])
                                for r in range(2)]) for k in range(2)])
    b2p = jnp.stack([jnp.stack([_rep8(b2[32 * k + r:32 * k + 32:2])
                                for r in range(2)]) for k in range(2)])

    p0, p1 = _deg_kernel(dst2)

    grid = (NP8 // BP8,)
    dinv8, xsw = pl.pallas_call(
        _prep_body, grid=grid,
        in_specs=[_pk_spec()] * 4,
        out_specs=[_pk_spec()] * 2,
        out_shape=[jax.ShapeDtypeStruct((NP8, 128), _f32),
                   jax.ShapeDtypeStruct((NP8, 128), _i32)],
    )(_c2p(p0), _c2p(p1), xe, xo)

    tp0, tp1 = _agg1_kernel(src2, dst2, _p2c(xsw))

    g0w, g1w = pl.pallas_call(
        _layer1_body, grid=grid,
        in_specs=[_pk_spec()] * 4 + [
            pl.BlockSpec((2, 2, 2, 128, 128), lambda i: (0, 0, 0, 0, 0)),
            pl.BlockSpec((2, 2, 128), lambda i: (0, 0, 0))],
        out_specs=[_pk_spec()] * 2,
        out_shape=[jax.ShapeDtypeStruct((NP8, 128), _i32)] * 2,
    )(_c2p(tp0), _c2p(tp1), xsw, dinv8, m1w, b1p)

    u0w, u1w = _agg2_kernel(src2, dst2, _p2c(g0w), _p2c(g1w))

    z8 = pl.pallas_call(
        _layer2_body, grid=grid,
        in_specs=[_pk_spec()] * 5 + [
            pl.BlockSpec((2, 2, 4, 128, 128), lambda i: (0, 0, 0, 0, 0)),
            pl.BlockSpec((2, 2, 128), lambda i: (0, 0, 0)),
            pl.BlockSpec((2, 2, 128, 128), lambda i: (0, 0, 0, 0))],
        out_specs=_pk_spec(),
        out_shape=jax.ShapeDtypeStruct((NP8, 128), _f32),
    )(_c2p(u0w), _c2p(u1w), g0w, g1w, dinv8, m2w, b2p, mzw)

    sums, cnts = _pool_kernel(z8, bpad)
    sums = sums.transpose(1, 0, 2).reshape(NSEG, 512)
    cnts = cnts.transpose(1, 0, 2).reshape(NSEG, 512)

    out = pl.pallas_call(
        _combine_body, grid=(1,),
        in_specs=[pl.BlockSpec((NSEG, 512), lambda i: (0, 0)),
                  pl.BlockSpec((NSEG, 512), lambda i: (0, 0)),
                  pl.BlockSpec((1, 1), lambda i: (0, 0))],
        out_specs=pl.BlockSpec((G, 1), lambda i: (0, 0)),
        out_shape=jax.ShapeDtypeStruct((G, 1), _f32),
    )(sums, cnts, lin_b.reshape(1, 1))
    return out


# speedup vs baseline: 2.4865x; 2.4865x over previous
import functools

import jax
import jax.numpy as jnp
from jax import lax
from jax.experimental import pallas as pl
from jax.experimental.pallas import tpu as pltpu
from jax.experimental.pallas import tpu_sc as plsc

N = 100000
E = 1600000
IN = 32
H = 64
G = 64

L = 16
F = 32
NTILES = 16
BN = 2048
NP = 49 * BN
NP4 = NP // 4
BP4 = BN // 4
EP = 1638400
CH = 1280
KSUB = CH // 128
RPT = NP // NTILES
STG = RPT // 16
NSEG = G + 1
PNT = NP // 32
PNR4 = PNT // 4

_f32 = jnp.float32
_bf16 = jnp.bfloat16
_i32 = jnp.int32


def _fill(ref, rows, vec):
    def body(i, _):
        ref[i] = vec
        return 0
    lax.fori_loop(0, rows, body, 0)


def _zero_accum(stage, accum, s):
    _fill(stage, STG, jnp.zeros((F,), _bf16))
    rz = s * RPT
    for r in range(16):
        pltpu.sync_copy(stage, accum.at[pl.ds(rz + r * STG, STG)])


def _write_out(stage, accum, out, s):
    rz = s * RPT
    for r in range(16):
        pltpu.sync_copy(accum.at[pl.ds(rz + r * STG, STG)], stage)
        pltpu.sync_copy(stage, out.at[pl.ds(rz + r * STG, STG)])


def _scan_edges(src2, dst2, table, sidx, didx, rows, accum,
                semg, sems, wid, nch):
    def chunk(j, _):
        rb = wid * (nch * KSUB) + j * KSUB
        pltpu.sync_copy(src2.at[pl.ds(rb, KSUB)], sidx)
        pltpu.sync_copy(dst2.at[pl.ds(rb, KSUB)], didx)
        gd = [pltpu.async_copy(table.at[sidx.at[k]],
                               rows.at[pl.ds(k * 128, 128)], semg)
              for k in range(KSUB)]
        sd = []
        for k in range(KSUB):
            gd[k].wait()
            sd.append(pltpu.async_copy(rows.at[pl.ds(k * 128, 128)],
                                       accum.at[didx.at[k]], sems, add=True))
        for d in sd:
            d.wait()
        return 0
    lax.fori_loop(0, nch, chunk, 0)


def _agg_pass(src2, dst2, table, out, sidx, didx, rows, stage, accum,
              semg, sems, s, wid, nch):
    _zero_accum(stage, accum, s)
    plsc.subcore_barrier()
    _scan_edges(src2, dst2, table, sidx, didx, rows, accum,
                semg, sems, wid, nch)
    plsc.subcore_barrier()
    _write_out(stage, accum, out, s)
    plsc.subcore_barrier()


_AGG_SCRATCH = [
    pltpu.VMEM((KSUB, 128), _i32),
    pltpu.VMEM((KSUB, 128), _i32),
    pltpu.VMEM((CH, F), _bf16),
    pltpu.VMEM((STG, F), _bf16),
    pltpu.VMEM_SHARED((NP, F), _bf16),
    pltpu.SemaphoreType.DMA,
    pltpu.SemaphoreType.DMA,
]


@functools.partial(
    pl.kernel,
    out_type=(jax.ShapeDtypeStruct((NP, F), _bf16),
              jax.ShapeDtypeStruct((NP, F), _bf16)),
    mesh=plsc.VectorSubcoreMesh(core_axis_name="c", subcore_axis_name="s"),
    scratch_types=[
        pltpu.VMEM((KSUB, 128), _i32),
        pltpu.VMEM((128, F), _bf16),
        pltpu.VMEM((STG, F), _bf16),
        pltpu.VMEM_SHARED((NP, F), _bf16),
        pltpu.SemaphoreType.DMA,
    ],
    compiler_params=pltpu.CompilerParams(use_tc_tiling_on_sc=False),
)
def _deg_kernel(dst2, o0, o1, didx, ones_v, stage, accum, sem):
    c = lax.axis_index("c")
    s = lax.axis_index("s")
    w = c * NTILES + s
    _fill(ones_v, 128, jnp.ones((F,), _bf16))
    _zero_accum(stage, accum, s)
    plsc.subcore_barrier()

    def chunk(j, _):
        rb = w * (EP // 32 // 128) + j * KSUB
        pltpu.sync_copy(dst2.at[pl.ds(rb, KSUB)], didx)
        sd = [pltpu.async_copy(ones_v, accum.at[didx.at[k]], sem, add=True)
              for k in range(KSUB)]
        for d in sd:
            d.wait()
        return 0
    lax.fori_loop(0, EP // 32 // CH, chunk, 0)
    plsc.subcore_barrier()

    @pl.when(c == 0)
    def _():
        _write_out(stage, accum, o0, s)

    @pl.when(c == 1)
    def _():
        _write_out(stage, accum, o1, s)


@functools.partial(
    pl.kernel,
    out_type=(jax.ShapeDtypeStruct((NP, F), _bf16),
              jax.ShapeDtypeStruct((NP, F), _bf16)),
    mesh=plsc.VectorSubcoreMesh(core_axis_name="c", subcore_axis_name="s"),
    scratch_types=list(_AGG_SCRATCH),
    compiler_params=pltpu.CompilerParams(use_tc_tiling_on_sc=False),
)
def _agg1_kernel(src2, dst2, xsb, t0, t1,
                 sidx, didx, rows, stage, accum, semg, sems):
    c = lax.axis_index("c")
    s = lax.axis_index("s")
    w = c * NTILES + s
    _zero_accum(stage, accum, s)
    plsc.subcore_barrier()
    _scan_edges(src2, dst2, xsb, sidx, didx, rows, accum,
                semg, sems, w, EP // 32 // CH)
    plsc.subcore_barrier()

    @pl.when(c == 0)
    def _():
        _write_out(stage, accum, t0, s)

    @pl.when(c == 1)
    def _():
        _write_out(stage, accum, t1, s)


@functools.partial(
    pl.kernel,
    out_type=(jax.ShapeDtypeStruct((NP, F), _bf16),
              jax.ShapeDtypeStruct((NP, F), _bf16)),
    mesh=plsc.VectorSubcoreMesh(core_axis_name="c", subcore_axis_name="s"),
    scratch_types=list(_AGG_SCRATCH),
    compiler_params=pltpu.CompilerParams(use_tc_tiling_on_sc=False),
)
def _agg2_kernel(src2, dst2, g0, g1, u0, u1,
                 sidx, didx, rows, stage, accum, semg, sems):
    c = lax.axis_index("c")
    s = lax.axis_index("s")

    @pl.when(c == 0)
    def _():
        _agg_pass(src2, dst2, g0, u0, sidx, didx, rows, stage, accum,
                  semg, sems, s, s, EP // NTILES // CH)

    @pl.when(c == 1)
    def _():
        _agg_pass(src2, dst2, g1, u1, sidx, didx, rows, stage, accum,
                  semg, sems, s, s, EP // NTILES // CH)


@functools.partial(
    pl.kernel,
    out_type=(jax.ShapeDtypeStruct((32, NSEG, L), _f32),
              jax.ShapeDtypeStruct((32, NSEG, L), _f32)),
    mesh=plsc.VectorSubcoreMesh(core_axis_name="c", subcore_axis_name="s"),
    scratch_types=[
        pltpu.VMEM((PNR4, 128), _f32),
        pltpu.VMEM((PNT,), _i32),
        pltpu.VMEM((NSEG, L), _f32),
        pltpu.VMEM((NSEG, L), _f32),
    ],
    compiler_params=pltpu.CompilerParams(use_tc_tiling_on_sc=False,
                                         needs_layout_passes=False),
)
def _pool_kernel(zp, bpad, sums_o, cnts_o, zv, bv, sacc, cacc):
    c = lax.axis_index("c")
    s = lax.axis_index("s")
    w = c * NTILES + s
    _fill(sacc, NSEG, jnp.zeros((L,), _f32))
    _fill(cacc, NSEG, jnp.zeros((L,), _f32))
    pltpu.sync_copy(zp.at[pl.ds(w * PNR4, PNR4)], zv)
    pltpu.sync_copy(bpad.at[pl.ds(w * PNT, PNT)], bv)
    lanes = lax.iota(_i32, L)
    ones16 = jnp.ones((L,), _f32)

    def grp(g, _):
        nloc = g * L + lanes
        ridx = nloc >> 2
        lidx = (nloc & 3) * F
        z16 = plsc.load_gather(zv, [ridx, lidx])
        b16 = bv[pl.ds(g * L, L)]
        plsc.addupdate_scatter(sacc, [b16, lanes], z16)
        plsc.addupdate_scatter(cacc, [b16, lanes], ones16)
        return 0
    lax.fori_loop(0, PNT // L, grp, 0)
    pltpu.sync_copy(sacc, sums_o.at[w])
    pltpu.sync_copy(cacc, cnts_o.at[w])


def _prep_body(p0, p1, x4, dinv_o, xsb_o):
    deg = 1.0 + p0[...].astype(_f32) + p1[...].astype(_f32)
    dinv4 = lax.rsqrt(deg)
    dinv_o[...] = dinv4
    xsb_o[...] = (x4[...] * dinv4).astype(_bf16)


def _layer1_body(tp0, tp1, xsb, dinv, m1b, b1p, g0_o, g1_o):
    dinv4 = dinv[...]
    t = tp0[...].astype(_f32) + tp1[...].astype(_f32)
    agg = (t + xsb[...].astype(_f32)) * dinv4
    m = m1b[...]
    b = b1p[...]
    for k, out in enumerate((g0_o, g1_o)):
        h = jnp.dot(agg, m[k], preferred_element_type=_f32) + b[k:k + 1, :]
        out[...] = (jnp.maximum(h, 0.0) * dinv4).astype(_bf16)


def _layer2_body(u0, u1, g0, g1, dinv, m2b, b2p, mzb, z_o):
    dinv4 = dinv[...]
    aggs = [(u[...].astype(_f32) + g[...].astype(_f32)) * dinv4
            for u, g in ((u0, g0), (u1, g1))]
    m = m2b[...]
    b = b2p[...]
    mzv = mzb[...]
    zp = jnp.zeros((BP4, 128), _f32)
    for k in range(2):
        h = b[k:k + 1, :]
        for s in range(2):
            h = h + jnp.dot(aggs[s], m[s, k], preferred_element_type=_f32)
        h = jnp.maximum(h, 0.0)
        zp = zp + jnp.dot(h, mzv[k], preferred_element_type=_f32)
    z_o[...] = zp


def _combine_body(sums, cnts, lb, out_o):
    ssum = jnp.sum(sums[...][:G, :], axis=1, keepdims=True)
    csum = jnp.sum(cnts[...][:G, :], axis=1, keepdims=True)
    out_o[...] = ssum / jnp.maximum(csum, 1.0) + lb[...]


def _pk_spec():
    return pl.BlockSpec((BP4, 128), lambda i: (i, 0))


def kernel(x, edge_index, batch, W1, b1, W2, b2, lin_W, lin_b):
    pad_row = jnp.arange(128, dtype=_i32)
    npad = (EP - E) // 128
    src2 = jnp.concatenate(
        [edge_index[0].reshape(E // 128, 128),
         jnp.broadcast_to(pad_row, (npad, 128))])
    dst2 = jnp.concatenate(
        [edge_index[1].reshape(E // 128, 128),
         jnp.broadcast_to(N + pad_row, (npad, 128))])
    bpad = jnp.concatenate([batch, jnp.full((NP - N,), G, _i32)])
    x4 = x.reshape(N // 4, 128)

    eye4 = jnp.eye(4, dtype=_f32)

    def bd4(w32):
        return jnp.einsum("pq,ij->piqj", eye4, w32).reshape(128, 128)

    m1b = jnp.stack([bd4(W1[:, 32 * k:32 * k + 32]) for k in range(2)])
    m2b = jnp.stack([jnp.stack([bd4(W2[32 * s:32 * s + 32, 32 * k:32 * k + 32])
                                for k in range(2)]) for s in range(2)])
    mzb = jnp.stack([jnp.einsum("pq,i,j->piqj", eye4,
                                lin_W[32 * k:32 * k + 32, 0],
                                jnp.ones((F,), _f32)).reshape(128, 128)
                     for k in range(2)])
    b1p = jnp.broadcast_to(b1.reshape(2, 1, F), (2, 4, F)).reshape(2, 128)
    b2p = jnp.broadcast_to(b2.reshape(2, 1, F), (2, 4, F)).reshape(2, 128)

    p0, p1 = _deg_kernel(dst2)

    grid = (NP4 // BP4,)
    dinv4, xsb = pl.pallas_call(
        _prep_body, grid=grid,
        in_specs=[_pk_spec()] * 3,
        out_specs=[_pk_spec()] * 2,
        out_shape=[jax.ShapeDtypeStruct((NP4, 128), _f32),
                   jax.ShapeDtypeStruct((NP4, 128), _bf16)],
    )(p0.reshape(NP4, 128), p1.reshape(NP4, 128), x4)

    tp0, tp1 = _agg1_kernel(src2, dst2, xsb.reshape(NP, F))

    g0b, g1b = pl.pallas_call(
        _layer1_body, grid=grid,
        in_specs=[_pk_spec()] * 4 + [
            pl.BlockSpec((2, 128, 128), lambda i: (0, 0, 0)),
            pl.BlockSpec((2, 128), lambda i: (0, 0))],
        out_specs=[_pk_spec()] * 2,
        out_shape=[jax.ShapeDtypeStruct((NP4, 128), _bf16)] * 2,
    )(tp0.reshape(NP4, 128), tp1.reshape(NP4, 128), xsb, dinv4, m1b, b1p)

    u0b, u1b = _agg2_kernel(src2, dst2, g0b.reshape(NP, F), g1b.reshape(NP, F))

    zp4 = pl.pallas_call(
        _layer2_body, grid=grid,
        in_specs=[_pk_spec()] * 5 + [
            pl.BlockSpec((2, 2, 128, 128), lambda i: (0, 0, 0, 0)),
            pl.BlockSpec((2, 128), lambda i: (0, 0)),
            pl.BlockSpec((2, 128, 128), lambda i: (0, 0, 0))],
        out_specs=_pk_spec(),
        out_shape=jax.ShapeDtypeStruct((NP4, 128), _f32),
    )(u0b.reshape(NP4, 128), u1b.reshape(NP4, 128), g0b, g1b, dinv4,
      m2b, b2p, mzb)

    sums, cnts = _pool_kernel(zp4, bpad)
    sums = sums.transpose(1, 0, 2).reshape(NSEG, 512)
    cnts = cnts.transpose(1, 0, 2).reshape(NSEG, 512)

    out = pl.pallas_call(
        _combine_body, grid=(1,),
        in_specs=[pl.BlockSpec((NSEG, 512), lambda i: (0, 0)),
                  pl.BlockSpec((NSEG, 512), lambda i: (0, 0)),
                  pl.BlockSpec((1, 1), lambda i: (0, 0))],
        out_specs=pl.BlockSpec((G, 1), lambda i: (0, 0)),
        out_shape=jax.ShapeDtypeStruct((G, 1), _f32),
    )(sums, cnts, lin_b.reshape(1, 1))
    return out
